# baseline (speedup 1.0000x reference)
import functools

import jax
import jax.numpy as jnp
from jax import lax
from jax.experimental import pallas as pl
from jax.experimental.pallas import tpu as pltpu
from jax.experimental.pallas import tpu_sc as plsc

_NC = 2
_NS = 16


def _sc_body(n_runs, topk, cb, sbs, kv_dim, rope_dim, ftab_cols, stab_cols,
             topk_hbm, ftab_hbm, stab_hbm, seq_hbm, kv_hbm, rope_hbm,
             zkv_hbm, zrope_hbm, out_rope, out_kv,
             topk_v, ftab_v, stab_v, seq_v, buf_kv, buf_rope):
    nw = _NC * _NS
    runs_per_w = n_runs // nw
    wid = lax.axis_index("s") * _NC + lax.axis_index("c")

    pltpu.sync_copy(topk_hbm, topk_v)
    pltpu.sync_copy(ftab_hbm, ftab_v)
    pltpu.sync_copy(stab_hbm, stab_v)
    pltpu.sync_copy(seq_hbm, seq_v)

    runs_per_cb = cb // sbs

    for k in range(runs_per_w):
        r = wid * runs_per_w + k
        b = r // topk
        t = r % topk
        idx = topk_v[pl.ds(r, 16)][0]
        src_blk = ftab_v[pl.ds(b * ftab_cols + idx // runs_per_cb, 16)][0]
        src = src_blk * cb + (idx % runs_per_cb) * sbs
        dst_blk = stab_v[pl.ds(b * stab_cols + t // runs_per_cb, 16)][0]
        dst = dst_blk * cb + (t % runs_per_cb) * sbs
        nv = jnp.clip(seq_v[pl.ds(b, 16)][0] - idx * sbs, 0, sbs)

        @pl.when(nv >= sbs)
        def _():
            pltpu.sync_copy(kv_hbm.at[pl.ds(src, sbs), :],
                            out_kv.at[pl.ds(dst, sbs), :])
            pltpu.sync_copy(rope_hbm.at[pl.ds(src, sbs), :],
                            out_rope.at[pl.ds(dst, sbs), :])

        @pl.when(nv <= 0)
        def _():
            pltpu.sync_copy(zkv_hbm, out_kv.at[pl.ds(dst, sbs), :])
            pltpu.sync_copy(zrope_hbm, out_rope.at[pl.ds(dst, sbs), :])

        @pl.when((nv > 0) & (nv < sbs))
        def _():
            pltpu.sync_copy(kv_hbm.at[pl.ds(src, sbs), :], buf_kv)
            pltpu.sync_copy(rope_hbm.at[pl.ds(src, sbs), :], buf_rope)

            zeros16 = jnp.zeros((16,), jnp.float32)

            def zrow(j, carry):
                for c in range(kv_dim // 16):
                    buf_kv[j, pl.ds(c * 16, 16)] = zeros16
                for c in range(rope_dim // 16):
                    buf_rope[j, pl.ds(c * 16, 16)] = zeros16
                return carry

            lax.fori_loop(nv, sbs, zrow, 0)
            pltpu.sync_copy(buf_kv, out_kv.at[pl.ds(dst, sbs), :])
            pltpu.sync_copy(buf_rope, out_rope.at[pl.ds(dst, sbs), :])


def kernel(selection_k_rope, selection_kv_cache, selection_kv_block_table,
           selection_kv_block_status, selection_topk_indices, full_k_rope,
           full_kv_cache, full_kv_block_table, full_kv_actual_seq,
           full_q_actual_seq, selection_topk_block_size):
    B, TOPK = selection_topk_indices.shape
    NFB, CB, KV_DIM = full_kv_cache.shape
    ROPE = full_k_rope.shape[-1]
    NSB = selection_kv_cache.shape[0]
    SBS = (NSB // B) * CB // TOPK
    N_RUNS = B * TOPK

    kv_flat = full_kv_cache.reshape(NFB * CB, KV_DIM)
    rope_flat = full_k_rope.reshape(NFB * CB, ROPE)
    pad = lambda a: jnp.pad(a.reshape(-1).astype(jnp.int32), (0, 16))
    topk_flat = pad(selection_topk_indices)
    ftab_flat = pad(full_kv_block_table)
    stab_flat = pad(selection_kv_block_table)
    seq = pad(full_kv_actual_seq)
    zkv = jnp.zeros((SBS, KV_DIM), jnp.float32)
    zrope = jnp.zeros((SBS, ROPE), jnp.float32)

    mesh = plsc.VectorSubcoreMesh(core_axis_name="c", subcore_axis_name="s",
                                  num_cores=_NC, num_subcores=_NS)
    body = functools.partial(_sc_body, N_RUNS, TOPK, CB, SBS, KV_DIM, ROPE,
                             full_kv_block_table.shape[1],
                             selection_kv_block_table.shape[1])
    out_rope, out_kv = pl.kernel(
        body,
        out_type=[
            jax.ShapeDtypeStruct((NSB * CB, ROPE), jnp.float32),
            jax.ShapeDtypeStruct((NSB * CB, KV_DIM), jnp.float32),
        ],
        mesh=mesh,
        scratch_types=[
            pltpu.VMEM((topk_flat.shape[0],), jnp.int32),
            pltpu.VMEM((ftab_flat.shape[0],), jnp.int32),
            pltpu.VMEM((stab_flat.shape[0],), jnp.int32),
            pltpu.VMEM((seq.shape[0],), jnp.int32),
            pltpu.VMEM((SBS, KV_DIM), jnp.float32),
            pltpu.VMEM((SBS, ROPE), jnp.float32),
        ],
    )(topk_flat, ftab_flat, stab_flat, seq, kv_flat, rope_flat, zkv, zrope)

    return (out_rope.reshape(NSB, CB, ROPE), out_kv.reshape(NSB, CB, KV_DIM))
